# baseline (speedup 1.0000x reference)
import functools

import jax
import jax.numpy as jnp
from jax import lax
from jax.experimental import pallas as pl
from jax.experimental.pallas import tpu as pltpu
from jax.experimental.pallas import tpu_sc as plsc

_INFO = plsc.get_sparse_core_info()
_NC, _NS = _INFO.num_cores, _INFO.num_subcores
_NW = _NC * _NS
_BB = 128


@functools.partial(jax.jit, static_argnums=(3, 4))
def _sc_gather(wt2, pair_t, sel_t, n_seq, d):
    n_b = pair_t.shape[1]
    assert n_b == _NW * _BB and d % 8 == 0 and n_seq % 4 == 0 and n_seq >= 12

    mesh = plsc.VectorSubcoreMesh(core_axis_name="c", subcore_axis_name="s")

    @functools.partial(
        pl.kernel,
        mesh=mesh,
        out_type=jax.ShapeDtypeStruct((n_seq, d, n_b), jnp.float32),
        scratch_types=[
            pltpu.VMEM((n_seq, _BB), jnp.int32),
            pltpu.VMEM((n_seq, _BB), jnp.int32),
            pltpu.VMEM((4, _BB, 2 * d), jnp.float32),
            pltpu.VMEM((d, _BB), jnp.float32),
            pltpu.SemaphoreType.DMA,
            pltpu.SemaphoreType.DMA,
        ],
        compiler_params=pltpu.CompilerParams(
            use_tc_tiling_on_sc=True, needs_layout_passes=False
        ),
    )
    def k(wt_hbm, pair_hbm, sel_hbm, out_hbm, pair_v, sel_v, buf_v, slab_v,
          gsem, ssem):
        w = lax.axis_index("s") * _NC + lax.axis_index("c")
        col0 = w * _BB
        pltpu.sync_copy(pair_hbm.at[:, pl.ds(col0, _BB)], pair_v)
        pltpu.sync_copy(sel_hbm.at[:, pl.ds(col0, _BB)], sel_v)

        row_ids = [lax.iota(jnp.int32, 16) + (16 * g) for g in range(8)]

        def gather_start(s, p):
            pltpu.async_copy(wt_hbm.at[pair_v.at[s]], buf_v.at[p], gsem)

        def gather_wait(p):
            pltpu.make_async_copy(
                wt_hbm.at[pair_v.at[0]], buf_v.at[p], gsem
            ).wait()

        def store_start(s):
            pltpu.async_copy(
                slab_v, out_hbm.at[s, :, pl.ds(col0, _BB)], ssem
            )

        def store_wait():
            pltpu.make_async_copy(
                slab_v, out_hbm.at[0, :, pl.ds(col0, _BB)], ssem
            ).wait()

        def transpose_select(s, p):
            bufp = buf_v.at[p]
            sel_vecs = tuple(
                sel_v[s, pl.ds(16 * g, 16)] for g in range(8)
            )

            def dbody(dk, cols):
                for u in range(4):
                    dd = dk * 4 + u
                    vals = [
                        plsc.load_gather(bufp, [row_ids[g], cols[g] + u])
                        for g in range(8)
                    ]
                    for g in range(8):
                        slab_v[dd, pl.ds(16 * g, 16)] = vals[g]
                return tuple(c + 4 for c in cols)

            lax.fori_loop(0, d // 4, dbody, sel_vecs)

        for b in range(4):
            gather_start(b, b)

        for b in range(4):
            gather_wait(b)
            if b > 0:
                store_wait()
            transpose_select(b, b)
            store_start(b)
            gather_start(b + 4, b)

        def unit_quad(t, carry):
            s0 = t * 4
            for b in range(4):
                s = s0 + b
                gather_wait(b)
                store_wait()
                transpose_select(s, b)
                store_start(s)
                gather_start(s + 4, b)
            return carry

        lax.fori_loop(1, n_seq // 4 - 1, unit_quad, 0)

        for b in range(4):
            s = n_seq - 4 + b
            gather_wait(b)
            store_wait()
            transpose_select(s, b)
            store_start(s)
        store_wait()

    return k(wt2, pair_t, sel_t)


def kernel(input_, weight):
    b, s = input_.shape
    v, d = weight.shape
    assert b == _NW * _BB and v % 2 == 0
    idx_t = input_.T.astype(jnp.int32)
    pair_t = idx_t >> 1
    sel_t = (idx_t & 1) * d
    wt2 = weight.reshape(v // 2, 2 * d)
    out_phys = _sc_gather(wt2, pair_t, sel_t, s, d)
    return out_phys.transpose(2, 0, 1)


# speedup vs baseline: 1.8981x; 1.8981x over previous
import functools

import jax
import jax.numpy as jnp
from jax import lax
from jax.experimental import pallas as pl
from jax.experimental.pallas import tpu as pltpu
from jax.experimental.pallas import tpu_sc as plsc

_INFO = plsc.get_sparse_core_info()
_NC, _NS = _INFO.num_cores, _INFO.num_subcores
_NW = _NC * _NS
_BB = 128


@functools.partial(jax.jit, static_argnums=(3, 4))
def _sc_gather(wt2, pair_t, sel_t, n_seq, d):
    n_b = pair_t.shape[1]
    assert n_b == _NW * _BB and d % 8 == 0 and n_seq >= 9

    mesh = plsc.VectorSubcoreMesh(core_axis_name="c", subcore_axis_name="s")

    @functools.partial(
        pl.kernel,
        mesh=mesh,
        out_type=jax.ShapeDtypeStruct((n_seq, d, n_b), jnp.float32),
        scratch_types=[
            pltpu.VMEM((n_seq, _BB), jnp.int32),
            pltpu.VMEM((n_seq, _BB), jnp.int32),
            pltpu.VMEM((3, _BB, 2 * d), jnp.float32),
            pltpu.VMEM((d, _BB + 2), jnp.float32),
            pltpu.SemaphoreType.DMA,
            pltpu.SemaphoreType.DMA,
        ],
        compiler_params=pltpu.CompilerParams(
            use_tc_tiling_on_sc=True, needs_layout_passes=False
        ),
    )
    def k(wt_hbm, pair_hbm, sel_hbm, out_hbm, pair_v, sel_v, buf_v, slab_v,
          gsem, ssem):
        w = lax.axis_index("s") * _NC + lax.axis_index("c")
        col0 = w * _BB
        pltpu.sync_copy(pair_hbm.at[:, pl.ds(col0, _BB)], pair_v)
        pltpu.sync_copy(sel_hbm.at[:, pl.ds(col0, _BB)], sel_v)

        row_ids = [lax.iota(jnp.int32, 16) + (16 * g) for g in range(8)]

        def gather_start(s, p):
            pltpu.async_copy(wt_hbm.at[pair_v.at[s]], buf_v.at[p], gsem)

        def gather_wait(p):
            pltpu.make_async_copy(
                wt_hbm.at[pair_v.at[0]], buf_v.at[p], gsem
            ).wait()

        def store_start(s):
            pltpu.async_copy(
                slab_v.at[:, pl.ds(0, _BB)],
                out_hbm.at[s, :, pl.ds(col0, _BB)],
                ssem,
            )

        def store_wait():
            pltpu.make_async_copy(
                slab_v.at[:, pl.ds(0, _BB)],
                out_hbm.at[0, :, pl.ds(col0, _BB)],
                ssem,
            ).wait()

        iota16 = lax.iota(jnp.int32, 16)
        dmask = d - 1

        def transpose_select(s, p):
            bufp = buf_v.at[p]
            base_cols = tuple(
                sel_v[s, pl.ds(16 * g, 16)] for g in range(8)
            )

            def dbody(dk, dvec):
                dv = dvec
                for _ in range(4):
                    vals = [
                        plsc.load_gather(bufp, [row_ids[g], base_cols[g] + dv])
                        for g in range(8)
                    ]
                    for g in range(8):
                        plsc.store_scatter(
                            slab_v, [dv, row_ids[g]], vals[g]
                        )
                    dv = (dv + 1) & dmask
                return dv

            lax.fori_loop(0, d // 4, dbody, iota16 & dmask)

        for b in range(3):
            gather_start(b, b)

        for b in range(3):
            gather_wait(b)
            if b > 0:
                store_wait()
            transpose_select(b, b)
            store_start(b)
            gather_start(b + 3, b)

        def unit_triad(t, carry):
            s0 = t * 3
            for b in range(3):
                s = s0 + b
                gather_wait(b)
                store_wait()
                transpose_select(s, b)
                store_start(s)
                gather_start(s + 3, b)
            return carry

        n_triads = n_seq // 3
        rem = n_seq - n_triads * 3
        lax.fori_loop(1, n_triads - 1, unit_triad, 0)

        for b in range(3):
            s = (n_triads - 1) * 3 + b
            gather_wait(b)
            store_wait()
            transpose_select(s, b)
            store_start(s)
            if rem and b < rem:
                gather_start(n_triads * 3 + b, b)
        for b in range(rem):
            s = n_triads * 3 + b
            gather_wait(b)
            store_wait()
            transpose_select(s, b)
            store_start(s)
        store_wait()

    return k(wt2, pair_t, sel_t)


def kernel(input_, weight):
    b, s = input_.shape
    v, d = weight.shape
    assert b == _NW * _BB and v % 2 == 0
    idx_t = input_.T.astype(jnp.int32)
    pair_t = idx_t >> 1
    sel_t = (idx_t & 1) * d
    wt2 = weight.reshape(v // 2, 2 * d)
    out_phys = _sc_gather(wt2, pair_t, sel_t, s, d)
    return out_phys.transpose(2, 0, 1)


# speedup vs baseline: 1.9941x; 1.0506x over previous
import functools

import jax
import jax.numpy as jnp
from jax import lax
from jax.experimental import pallas as pl
from jax.experimental.pallas import tpu as pltpu
from jax.experimental.pallas import tpu_sc as plsc

_INFO = plsc.get_sparse_core_info()
_NC, _NS = _INFO.num_cores, _INFO.num_subcores
_NW = _NC * _NS
_BB = 128


@functools.partial(jax.jit, static_argnums=(3, 4))
def _sc_gather(wt2, pair_t, sel_t, n_seq, d):
    n_b = pair_t.shape[1]
    assert n_b == _NW * _BB and d % 8 == 0 and n_seq >= 9

    mesh = plsc.VectorSubcoreMesh(core_axis_name="c", subcore_axis_name="s")

    @functools.partial(
        pl.kernel,
        mesh=mesh,
        out_type=jax.ShapeDtypeStruct((n_seq, d, n_b), jnp.float32),
        scratch_types=[
            pltpu.VMEM((n_seq, _BB), jnp.int32),
            pltpu.VMEM((n_seq, _BB), jnp.int32),
            pltpu.VMEM((3, _BB, 2 * d), jnp.float32),
            pltpu.VMEM((d, _BB), jnp.float32),
            pltpu.SemaphoreType.DMA,
            pltpu.SemaphoreType.DMA,
        ],
        compiler_params=pltpu.CompilerParams(
            use_tc_tiling_on_sc=True, needs_layout_passes=False
        ),
    )
    def k(wt_hbm, pair_hbm, sel_hbm, out_hbm, pair_v, sel_v, buf_v, slab_v,
          gsem, ssem):
        w = lax.axis_index("s") * _NC + lax.axis_index("c")
        col0 = w * _BB
        pltpu.sync_copy(pair_hbm.at[:, pl.ds(col0, _BB)], pair_v)
        pltpu.sync_copy(sel_hbm.at[:, pl.ds(col0, _BB)], sel_v)

        row_ids = [lax.iota(jnp.int32, 16) + (16 * g) for g in range(8)]

        def gather_start(s, p):
            pltpu.async_copy(wt_hbm.at[pair_v.at[s]], buf_v.at[p], gsem)

        def gather_wait(p):
            pltpu.make_async_copy(
                wt_hbm.at[pair_v.at[0]], buf_v.at[p], gsem
            ).wait()

        def store_start(s):
            pltpu.async_copy(
                slab_v, out_hbm.at[s, :, pl.ds(col0, _BB)], ssem
            )

        def store_wait():
            pltpu.make_async_copy(
                slab_v, out_hbm.at[0, :, pl.ds(col0, _BB)], ssem
            ).wait()

        iota16 = lax.iota(jnp.int32, 16)
        dmask = d - 1

        def transpose_select(s, p):
            bufp = buf_v.at[p]
            base_cols = tuple(
                sel_v[s, pl.ds(16 * g, 16)] for g in range(8)
            )

            def dbody(dk, dvec):
                dv = dvec
                for _ in range(4):
                    vals = [
                        plsc.load_gather(bufp, [row_ids[g], base_cols[g] + dv])
                        for g in range(8)
                    ]
                    for g in range(8):
                        plsc.store_scatter(
                            slab_v, [dv, row_ids[g]], vals[g]
                        )
                    dv = (dv + 1) & dmask
                return dv

            lax.fori_loop(0, d // 4, dbody, iota16 & dmask)

        for b in range(3):
            gather_start(b, b)

        for b in range(3):
            gather_wait(b)
            if b > 0:
                store_wait()
            transpose_select(b, b)
            store_start(b)
            gather_start(b + 3, b)

        def unit_triad(t, carry):
            s0 = t * 3
            for b in range(3):
                s = s0 + b
                gather_wait(b)
                store_wait()
                transpose_select(s, b)
                store_start(s)
                gather_start(s + 3, b)
            return carry

        n_triads = n_seq // 3
        rem = n_seq - n_triads * 3
        lax.fori_loop(1, n_triads - 1, unit_triad, 0)

        for b in range(3):
            s = (n_triads - 1) * 3 + b
            gather_wait(b)
            store_wait()
            transpose_select(s, b)
            store_start(s)
            if rem and b < rem:
                gather_start(n_triads * 3 + b, b)
        for b in range(rem):
            s = n_triads * 3 + b
            gather_wait(b)
            store_wait()
            transpose_select(s, b)
            store_start(s)
        store_wait()

    return k(wt2, pair_t, sel_t)


def kernel(input_, weight):
    b, s = input_.shape
    v, d = weight.shape
    assert b == _NW * _BB and v % 2 == 0
    idx_t = input_.T.astype(jnp.int32)
    pair_t = idx_t >> 1
    sel_t = (idx_t & 1) * d
    wt2 = weight.reshape(v // 2, 2 * d)
    out_phys = _sc_gather(wt2, pair_t, sel_t, s, d)
    return out_phys.transpose(2, 0, 1)
